# Initial kernel scaffold; baseline (speedup 1.0000x reference)
import jax
import jax.numpy as jnp
from jax.experimental import pallas as pl


def kernel(z, weight):
    raise NotImplementedError("write your pallas kernel here")


# speedup vs baseline: 1.0117x; 1.0117x over previous
import functools

import jax
import jax.numpy as jnp
from jax import lax
from jax.experimental import pallas as pl
from jax.experimental.pallas import tpu as pltpu
from jax.experimental.pallas import tpu_sc as plsc

N, K, D = 8192, 8192, 64
BN = 256


def _argmin_body(z_ref, w_ref, idx_ref):
    z = z_ref[...]
    w = w_ref[...]
    zz = jnp.sum(z * z, axis=1, keepdims=True)
    ww = jnp.sum(w * w, axis=1)
    scores = (
        zz
        + ww[None, :]
        - 2.0 * lax.dot_general(z, w, (((1,), (1,)), ((), ())),
                                preferred_element_type=jnp.float32)
    )
    idx_ref[...] = jnp.argmin(scores, axis=1).astype(jnp.int32)[:, None]


_argmin_call = pl.pallas_call(
    _argmin_body,
    grid=(N // BN,),
    in_specs=[
        pl.BlockSpec((BN, D), lambda i: (i, 0)),
        pl.BlockSpec((K, D), lambda i: (0, 0)),
    ],
    out_specs=pl.BlockSpec((BN, 1), lambda i: (i, 0)),
    out_shape=jax.ShapeDtypeStruct((N, 1), jnp.int32),
)


_info = plsc.get_sparse_core_info()
_NW = _info.num_cores * _info.num_subcores
_BPW = N // _NW


@functools.partial(
    pl.kernel,
    mesh=plsc.VectorSubcoreMesh(core_axis_name="c", subcore_axis_name="s"),
    out_type=jax.ShapeDtypeStruct((N, D), jnp.float32),
    compiler_params=pltpu.CompilerParams(use_tc_tiling_on_sc=False),
    scratch_types=[
        pltpu.VMEM((_BPW,), jnp.int32),
        pltpu.VMEM((_BPW, D), jnp.float32),
        pltpu.SemaphoreType.DMA,
    ],
)
def _sc_gather(w_hbm, idx_hbm, out_hbm, idx_v, rows_v, sem):
    wid = lax.axis_index("s") * _info.num_cores + lax.axis_index("c")
    base = wid * _BPW
    pltpu.sync_copy(idx_hbm.at[pl.ds(base, _BPW)], idx_v)
    pltpu.async_copy(w_hbm.at[idx_v], rows_v, sem).wait()
    pltpu.sync_copy(rows_v, out_hbm.at[pl.ds(base, _BPW)])


@jax.jit
def kernel(z, weight):
    indices = _argmin_call(z, weight).reshape(N)
    quantized = _sc_gather(weight, indices)
    return indices, quantized
